# Initial kernel scaffold; baseline (speedup 1.0000x reference)
import jax
import jax.numpy as jnp
from jax.experimental import pallas as pl


def kernel(x, edge_index, W1, b1, W2, b2):
    raise NotImplementedError("write your pallas kernel here")


# speedup vs baseline: 1.6256x; 1.6256x over previous
import functools

import jax
import jax.numpy as jnp
from jax import lax
from jax.experimental import pallas as pl
from jax.experimental.pallas import tpu as pltpu
from jax.experimental.pallas import tpu_sc as plsc

N_NODES = 10000
N_EDGES = 160000
IN_CH = 256
HID_CH = 512
OUT_CH = 256

NC = 2
NS = 16
LANES = 16
FAR = 1 << 30


def _make_scatter(n_nodes: int, n_edges: int, d: int, chunk_sizes, K: int):
    s = d // 128
    assert s * 128 == d and K % LANES == 0 and K <= 128
    rows_per_core = n_nodes // NC
    assert sum(chunk_sizes) == rows_per_core
    assert all(c % 8 == 0 for c in chunk_sizes)
    max_ch = max(chunk_sizes)
    ept = n_edges // NS
    assert ept * NS == n_edges and ept % LANES == 0
    ept_pad = -(-ept // K) * K
    nb = ept_pad // K

    mesh = plsc.VectorSubcoreMesh(core_axis_name="c", subcore_axis_name="s")

    @functools.partial(
        pl.kernel,
        out_type=jax.ShapeDtypeStruct((n_nodes, s, 128), jnp.float32),
        mesh=mesh,
        scratch_types=[
            pltpu.VMEM_SHARED((max_ch + 8, s, 128), jnp.float32),
            pltpu.VMEM((ept_pad,), jnp.int32),
            pltpu.VMEM((ept_pad,), jnp.int32),
            pltpu.VMEM((K,), jnp.int32),
            pltpu.VMEM((1, K), jnp.int32),
            pltpu.VMEM((K, s, 128), jnp.float32),
            pltpu.SemaphoreType.DMA,
        ],
    )
    def scatter(h_hbm, src_hbm, dst_hbm, out_hbm,
                acc, src_v, dst_v, sidx, didx, rows, sem):
        cid = lax.axis_index("c")
        sid = lax.axis_index("s")
        ebase = sid * ept
        pltpu.sync_copy(src_hbm.at[pl.ds(ebase, ept)], src_v.at[pl.ds(0, ept)])
        pltpu.sync_copy(dst_hbm.at[pl.ds(ebase, ept)], dst_v.at[pl.ds(0, ept)])
        for j in range(ept, ept_pad, LANES):
            src_v[pl.ds(j, LANES)] = jnp.zeros((LANES,), jnp.int32)
            dst_v[pl.ds(j, LANES)] = jnp.full((LANES,), FAR, jnp.int32)

        for core in range(NC):
            chunk_lo = core * rows_per_core
            for ch in chunk_sizes:
                row_lo = chunk_lo
                chunk_lo += ch

                @pl.when(cid == core)
                def _(row_lo=row_lo, ch=ch):
                    rpt = -(-(ch // 8) // NS) * 8
                    r0 = pl.multiple_of(jnp.minimum(sid * rpt, ch - rpt), 8)

                    pltpu.sync_copy(h_hbm.at[pl.ds(row_lo + r0, rpt)],
                                    acc.at[pl.ds(r0, rpt)])
                    plsc.subcore_barrier()

                    def bbody(b, carry):
                        base = pl.multiple_of(b * K, K)
                        for j in range(0, K, LANES):
                            dvec = dst_v[pl.ds(base + j, LANES)]
                            svec = src_v[pl.ds(base + j, LANES)]
                            dloc = dvec - row_lo
                            m = (dloc >= 0) & (dloc < ch)
                            didx[0, pl.ds(j, LANES)] = jnp.where(
                                m, dloc, max_ch)
                            sidx[pl.ds(j, LANES)] = svec
                        pltpu.async_copy(h_hbm.at[sidx], rows, sem).wait()
                        pltpu.sync_copy(rows, acc.at[didx.at[0]], add=True)
                        return carry

                    lax.fori_loop(0, nb, bbody, jnp.int32(0))
                    plsc.subcore_barrier()

                    pltpu.sync_copy(acc.at[pl.ds(r0, rpt)],
                                    out_hbm.at[pl.ds(row_lo + r0, rpt)])
                    plsc.subcore_barrier()

    return scatter


_CHUNKS = [2040, 2040, 920]
_scatter_hid = _make_scatter(N_NODES, N_EDGES, HID_CH, _CHUNKS, K=32)
_scatter_out = _make_scatter(N_NODES, N_EDGES, OUT_CH, _CHUNKS, K=64)


_BM = 1000


def _mm1_body(x_ref, w_ref, o_ref):
    o_ref[...] = lax.dot_general(
        x_ref[...], w_ref[...], (((1,), (1,)), ((), ())),
        preferred_element_type=jnp.float32)


def _mm1(x, w1):
    n = x.shape[0]
    return pl.pallas_call(
        _mm1_body,
        grid=(n // _BM,),
        in_specs=[
            pl.BlockSpec((_BM, IN_CH), lambda i: (i, 0)),
            pl.BlockSpec((HID_CH, IN_CH), lambda i: (0, 0)),
        ],
        out_specs=pl.BlockSpec((_BM, HID_CH), lambda i: (i, 0)),
        out_shape=jax.ShapeDtypeStruct((n, HID_CH), jnp.float32),
    )(x, w1)


def _mm2_body(s_ref, b_ref, w_ref, o_ref):
    g = jnp.maximum(s_ref[...] + b_ref[...], 0.0)
    o_ref[...] = lax.dot_general(
        g, w_ref[...], (((1,), (1,)), ((), ())),
        preferred_element_type=jnp.float32)


def _mm2(s1, b1, w2):
    n = s1.shape[0]
    return pl.pallas_call(
        _mm2_body,
        grid=(n // _BM,),
        in_specs=[
            pl.BlockSpec((_BM, HID_CH), lambda i: (i, 0)),
            pl.BlockSpec((1, HID_CH), lambda i: (0, 0)),
            pl.BlockSpec((OUT_CH, HID_CH), lambda i: (0, 0)),
        ],
        out_specs=pl.BlockSpec((_BM, OUT_CH), lambda i: (i, 0)),
        out_shape=jax.ShapeDtypeStruct((n, OUT_CH), jnp.float32),
    )(s1, b1.reshape(1, HID_CH), w2)


def _softmax_body(s_ref, b_ref, o_ref):
    z = s_ref[...] + b_ref[...]
    z = z - jnp.max(z, axis=-1, keepdims=True)
    e = jnp.exp(z)
    o_ref[...] = e / jnp.sum(e, axis=-1, keepdims=True)


def _softmax(s2, b2):
    n = s2.shape[0]
    return pl.pallas_call(
        _softmax_body,
        grid=(n // _BM,),
        in_specs=[
            pl.BlockSpec((_BM, OUT_CH), lambda i: (i, 0)),
            pl.BlockSpec((1, OUT_CH), lambda i: (0, 0)),
        ],
        out_specs=pl.BlockSpec((_BM, OUT_CH), lambda i: (i, 0)),
        out_shape=jax.ShapeDtypeStruct((n, OUT_CH), jnp.float32),
    )(s2, b2.reshape(1, OUT_CH))


def kernel(x, edge_index, W1, b1, W2, b2):
    src = edge_index[0].astype(jnp.int32)
    dst = edge_index[1].astype(jnp.int32)
    h1 = _mm1(x, W1)
    s1 = _scatter_hid(h1.reshape(N_NODES, HID_CH // 128, 128), src, dst)
    s1 = s1.reshape(N_NODES, HID_CH)
    h2 = _mm2(s1, b1, W2)
    s2 = _scatter_out(h2.reshape(N_NODES, OUT_CH // 128, 128), src, dst)
    s2 = s2.reshape(N_NODES, OUT_CH)
    return _softmax(s2, b2)


# speedup vs baseline: 2.9010x; 1.7845x over previous
import functools

import jax
import jax.numpy as jnp
from jax import lax
from jax.experimental import pallas as pl
from jax.experimental.pallas import tpu as pltpu
from jax.experimental.pallas import tpu_sc as plsc

N_NODES = 10000
N_EDGES = 160000
IN_CH = 256
HID_CH = 512
OUT_CH = 256

NC = 2
NS = 16
LANES = 16
FAR = 1 << 30


def _make_scatter(n_nodes: int, n_edges: int, d: int, chunk_sizes, K: int):
    s = d // 128
    assert s * 128 == d and K % LANES == 0 and K <= 128
    rows_per_core = n_nodes // NC
    assert sum(chunk_sizes) == rows_per_core
    assert all(c % 8 == 0 for c in chunk_sizes)
    max_ch = max(chunk_sizes)
    ept = n_edges // NS
    assert ept * NS == n_edges and ept % LANES == 0
    ept_pad = -(-ept // (2 * K)) * (2 * K)
    nb = ept_pad // K

    mesh = plsc.VectorSubcoreMesh(core_axis_name="c", subcore_axis_name="s")

    @functools.partial(
        pl.kernel,
        out_type=jax.ShapeDtypeStruct((n_nodes, s, 128), jnp.float32),
        mesh=mesh,
        scratch_types=[
            pltpu.VMEM_SHARED((max_ch + 8, s, 128), jnp.float32),
            pltpu.VMEM((ept_pad,), jnp.int32),
            pltpu.VMEM((ept_pad,), jnp.int32),
            pltpu.VMEM((K,), jnp.int32),
            pltpu.VMEM((K,), jnp.int32),
            pltpu.VMEM((1, K), jnp.int32),
            pltpu.VMEM((1, K), jnp.int32),
            pltpu.VMEM((K, s, 128), jnp.float32),
            pltpu.VMEM((K, s, 128), jnp.float32),
            pltpu.SemaphoreType.DMA,
            pltpu.SemaphoreType.DMA,
        ],
    )
    def scatter(h_hbm, src_hbm, dst_hbm, out_hbm,
                acc, src_v, dst_v, sidx0, sidx1, didx0, didx1,
                rows0, rows1, sem0, sem1):
        sidx = (sidx0, sidx1)
        didx = (didx0, didx1)
        rows = (rows0, rows1)
        sem = (sem0, sem1)
        cid = lax.axis_index("c")
        sid = lax.axis_index("s")
        ebase = sid * ept
        pltpu.sync_copy(src_hbm.at[pl.ds(ebase, ept)], src_v.at[pl.ds(0, ept)])
        pltpu.sync_copy(dst_hbm.at[pl.ds(ebase, ept)], dst_v.at[pl.ds(0, ept)])
        for j in range(ept, ept_pad, LANES):
            src_v[pl.ds(j, LANES)] = jnp.zeros((LANES,), jnp.int32)
            dst_v[pl.ds(j, LANES)] = jnp.full((LANES,), FAR, jnp.int32)

        for core in range(NC):
            chunk_lo = core * rows_per_core
            for ch in chunk_sizes:
                row_lo = chunk_lo
                chunk_lo += ch

                @pl.when(cid == core)
                def _(row_lo=row_lo, ch=ch):
                    rpt = -(-(ch // 8) // NS) * 8
                    r0 = pl.multiple_of(jnp.minimum(sid * rpt, ch - rpt), 8)

                    pltpu.sync_copy(h_hbm.at[pl.ds(row_lo + r0, rpt)],
                                    acc.at[pl.ds(r0, rpt)])
                    plsc.subcore_barrier()

                    def bbody(g, carry):
                        descs = []
                        for p in range(2):
                            base = pl.multiple_of((2 * g + p) * K, K)
                            for j in range(0, K, LANES):
                                dvec = dst_v[pl.ds(base + j, LANES)]
                                svec = src_v[pl.ds(base + j, LANES)]
                                dloc = dvec - row_lo
                                m = (dloc >= 0) & (dloc < ch)
                                didx[p][0, pl.ds(j, LANES)] = jnp.where(
                                    m, dloc, max_ch)
                                sidx[p][pl.ds(j, LANES)] = svec
                            descs.append(pltpu.async_copy(
                                h_hbm.at[sidx[p]], rows[p], sem[p]))
                        for p in range(2):
                            descs[p].wait()
                            pltpu.sync_copy(rows[p], acc.at[didx[p].at[0]],
                                            add=True)
                        return carry

                    lax.fori_loop(0, nb // 2, bbody, jnp.int32(0))
                    plsc.subcore_barrier()

                    pltpu.sync_copy(acc.at[pl.ds(r0, rpt)],
                                    out_hbm.at[pl.ds(row_lo + r0, rpt)])
                    plsc.subcore_barrier()

    return scatter


_scatter_hid = _make_scatter(N_NODES, N_EDGES, HID_CH, [2920, 2080], K=16)
_scatter_out = _make_scatter(N_NODES, N_EDGES, OUT_CH, [5000], K=32)


_BM = 1000


def _mm1_body(x_ref, w_ref, o_ref):
    o_ref[...] = lax.dot_general(
        x_ref[...], w_ref[...], (((1,), (1,)), ((), ())),
        preferred_element_type=jnp.float32)


def _mm1(x, w1):
    n = x.shape[0]
    return pl.pallas_call(
        _mm1_body,
        grid=(n // _BM,),
        in_specs=[
            pl.BlockSpec((_BM, IN_CH), lambda i: (i, 0)),
            pl.BlockSpec((HID_CH, IN_CH), lambda i: (0, 0)),
        ],
        out_specs=pl.BlockSpec((_BM, HID_CH), lambda i: (i, 0)),
        out_shape=jax.ShapeDtypeStruct((n, HID_CH), jnp.float32),
    )(x, w1)


def _mm2_body(s_ref, b_ref, w_ref, o_ref):
    g = jnp.maximum(s_ref[...] + b_ref[...], 0.0)
    o_ref[...] = lax.dot_general(
        g, w_ref[...], (((1,), (1,)), ((), ())),
        preferred_element_type=jnp.float32)


def _mm2(s1, b1, w2):
    n = s1.shape[0]
    return pl.pallas_call(
        _mm2_body,
        grid=(n // _BM,),
        in_specs=[
            pl.BlockSpec((_BM, HID_CH), lambda i: (i, 0)),
            pl.BlockSpec((1, HID_CH), lambda i: (0, 0)),
            pl.BlockSpec((OUT_CH, HID_CH), lambda i: (0, 0)),
        ],
        out_specs=pl.BlockSpec((_BM, OUT_CH), lambda i: (i, 0)),
        out_shape=jax.ShapeDtypeStruct((n, OUT_CH), jnp.float32),
    )(s1, b1.reshape(1, HID_CH), w2)


def _softmax_body(s_ref, b_ref, o_ref):
    z = s_ref[...] + b_ref[...]
    z = z - jnp.max(z, axis=-1, keepdims=True)
    e = jnp.exp(z)
    o_ref[...] = e / jnp.sum(e, axis=-1, keepdims=True)


def _softmax(s2, b2):
    n = s2.shape[0]
    return pl.pallas_call(
        _softmax_body,
        grid=(n // _BM,),
        in_specs=[
            pl.BlockSpec((_BM, OUT_CH), lambda i: (i, 0)),
            pl.BlockSpec((1, OUT_CH), lambda i: (0, 0)),
        ],
        out_specs=pl.BlockSpec((_BM, OUT_CH), lambda i: (i, 0)),
        out_shape=jax.ShapeDtypeStruct((n, OUT_CH), jnp.float32),
    )(s2, b2.reshape(1, OUT_CH))


def kernel(x, edge_index, W1, b1, W2, b2):
    src = edge_index[0].astype(jnp.int32)
    dst = edge_index[1].astype(jnp.int32)
    h1 = _mm1(x, W1)
    s1 = _scatter_hid(h1.reshape(N_NODES, HID_CH // 128, 128), src, dst)
    s1 = s1.reshape(N_NODES, HID_CH)
    h2 = _mm2(s1, b1, W2)
    s2 = _scatter_out(h2.reshape(N_NODES, OUT_CH // 128, 128), src, dst)
    s2 = s2.reshape(N_NODES, OUT_CH)
    return _softmax(s2, b2)
